# Initial kernel scaffold; baseline (speedup 1.0000x reference)
import jax
import jax.numpy as jnp
from jax.experimental import pallas as pl


def kernel(x, edge_index, W0, b0, Ws, W1, b1):
    raise NotImplementedError("write your pallas kernel here")


# speedup vs baseline: 5.1315x; 5.1315x over previous
import functools

import numpy as np
import jax
import jax.numpy as jnp
from jax import lax
from jax.experimental import pallas as pl
from jax.experimental.pallas import tpu as pltpu
from jax.experimental.pallas import tpu_sc as plsc

_N = 10000
_E = 320000
_DIN = 128
_H = 64
_HP = 128
_C = 40
_L = 8
_ALPHA = 0.1
_THETA = 0.5

_NC = 2
_NS = 16
_NW = _NC * _NS
_CH = 128
_WCH = 80
_EPAD = _NW * _WCH * _CH
_RPT = 632
_NP = _RPT * _NS
_GRID = _NS

_RCHUNKS = (0, 128, 256, 384, 504)


def _sc_mesh():
    return plsc.VectorSubcoreMesh(core_axis_name="c", subcore_axis_name="s")


def _fill_iota(buf, base):
    for i in range(_CH // 16):
        buf[pl.ds(16 * i, 16)] = base + 16 * i + lax.iota(jnp.int32, 16)


def _spmm_call(g, cols2, rows2):
    @functools.partial(
        pl.kernel,
        out_type=[
            jax.ShapeDtypeStruct((_NP, _HP), jnp.float32),
            jax.ShapeDtypeStruct((_NP, _HP), jnp.float32),
        ],
        mesh=_sc_mesh(),
        scratch_types=[
            pltpu.VMEM((_WCH, _CH), jnp.int32),
            pltpu.VMEM((_WCH, _CH), jnp.int32),
            pltpu.VMEM((_CH, _HP), jnp.float32),
            pltpu.VMEM((_CH,), jnp.int32),
            pltpu.VMEM_SHARED((_NP, _HP), jnp.float32),
        ],
    )
    def run(g_hbm, cols_hbm, rows_hbm, outA_hbm, outB_hbm,
            cidx, ridx, gbuf, ibuf, acc):
        c = lax.axis_index("c")
        s = lax.axis_index("s")
        w = c * _NS + s
        rb = pl.multiple_of(s * _RPT, 8)

        _fill_iota(ibuf, w * _WCH)
        pltpu.sync_copy(cols_hbm.at[ibuf.at[pl.ds(0, _WCH)]], cidx)
        pltpu.sync_copy(rows_hbm.at[ibuf.at[pl.ds(0, _WCH)]], ridx)

        for start in _RCHUNKS:
            _fill_iota(ibuf, rb + start)
            pltpu.sync_copy(g_hbm.at[ibuf], gbuf)
            pltpu.sync_copy(gbuf, acc.at[ibuf])
        plsc.subcore_barrier()

        def step(j, carry):
            pltpu.sync_copy(g_hbm.at[cidx.at[j]], gbuf)
            pltpu.sync_copy(gbuf, acc.at[ridx.at[j]], add=True)
            return carry

        lax.fori_loop(0, _WCH, step, 0)
        plsc.subcore_barrier()

        for start in _RCHUNKS:
            b = pl.multiple_of(rb + start, 8)
            pltpu.sync_copy(acc.at[pl.ds(b, _CH)], gbuf)

            @pl.when(c == 0)
            def _():
                pltpu.sync_copy(gbuf, outA_hbm.at[pl.ds(b, _CH)])

            @pl.when(c == 1)
            def _():
                pltpu.sync_copy(gbuf, outB_hbm.at[pl.ds(b, _CH)])

    return run(g, cols2, rows2)


def _pre_call(x, W0, b0, d0, d1):
    def body(x_ref, w_ref, b_ref, d0_ref, d1_ref, x0_ref, g_ref, dis_ref):
        deg = d0_ref[...] + d1_ref[...] - 1.0
        dis = lax.rsqrt(deg)
        h = jnp.dot(x_ref[...], w_ref[...], preferred_element_type=jnp.float32)
        h = jnp.maximum(h + b_ref[...], 0.0)
        g = dis * h
        x0_ref[...] = h
        g_ref[...] = jnp.concatenate(
            [g, jnp.zeros((_RPT, _HP - _H), jnp.float32)], axis=1)
        dis_ref[...] = dis

    return pl.pallas_call(
        body,
        grid=(_GRID,),
        in_specs=[
            pl.BlockSpec((_RPT, _DIN), lambda i: (i, 0)),
            pl.BlockSpec((_DIN, _H), lambda i: (0, 0)),
            pl.BlockSpec((1, _H), lambda i: (0, 0)),
            pl.BlockSpec((_RPT, 1), lambda i: (i, 0)),
            pl.BlockSpec((_RPT, 1), lambda i: (i, 0)),
        ],
        out_specs=[
            pl.BlockSpec((_RPT, _H), lambda i: (i, 0)),
            pl.BlockSpec((_RPT, _HP), lambda i: (i, 0)),
            pl.BlockSpec((_RPT, 1), lambda i: (i, 0)),
        ],
        out_shape=[
            jax.ShapeDtypeStruct((_NP, _H), jnp.float32),
            jax.ShapeDtypeStruct((_NP, _HP), jnp.float32),
            jax.ShapeDtypeStruct((_NP, 1), jnp.float32),
        ],
    )(x, W0, b0, d0, d1)


def _mid_call(p0, p1, g, x0, dis, Wl, beta):
    def body(p0_ref, p1_ref, g_ref, x0_ref, dis_ref, w_ref, beta_ref, go_ref):
        beta_v = beta_ref[...]
        hp0 = (p0_ref[...] + p1_ref[...] - g_ref[...])[:, :_H]
        hp = dis_ref[...] * hp0
        hpb = (1.0 - _ALPHA) * hp + _ALPHA * x0_ref[...]
        m = jnp.dot(hpb, w_ref[...], preferred_element_type=jnp.float32)
        h = jnp.maximum((1.0 - beta_v) * hpb + beta_v * m, 0.0)
        g = dis_ref[...] * h
        go_ref[...] = jnp.concatenate(
            [g, jnp.zeros((_RPT, _HP - _H), jnp.float32)], axis=1)

    return pl.pallas_call(
        body,
        grid=(_GRID,),
        in_specs=[
            pl.BlockSpec((_RPT, _HP), lambda i: (i, 0)),
            pl.BlockSpec((_RPT, _HP), lambda i: (i, 0)),
            pl.BlockSpec((_RPT, _HP), lambda i: (i, 0)),
            pl.BlockSpec((_RPT, _H), lambda i: (i, 0)),
            pl.BlockSpec((_RPT, 1), lambda i: (i, 0)),
            pl.BlockSpec((_H, _H), lambda i: (0, 0)),
            pl.BlockSpec((1, 1), lambda i: (0, 0)),
        ],
        out_specs=pl.BlockSpec((_RPT, _HP), lambda i: (i, 0)),
        out_shape=jax.ShapeDtypeStruct((_NP, _HP), jnp.float32),
    )(p0, p1, g, x0, dis, Wl, beta)


def _head_call(g, dis, W1, b1):
    def body(g_ref, dis_ref, w_ref, b_ref, o_ref):
        h = g_ref[...][:, :_H] / dis_ref[...]
        z = jnp.dot(h, w_ref[...], preferred_element_type=jnp.float32)
        z = z + b_ref[...]
        zm = z - jnp.max(z, axis=1, keepdims=True)
        o_ref[...] = zm - jnp.log(jnp.sum(jnp.exp(zm), axis=1, keepdims=True))

    return pl.pallas_call(
        body,
        grid=(_GRID,),
        in_specs=[
            pl.BlockSpec((_RPT, _HP), lambda i: (i, 0)),
            pl.BlockSpec((_RPT, 1), lambda i: (i, 0)),
            pl.BlockSpec((_H, _C), lambda i: (0, 0)),
            pl.BlockSpec((1, _C), lambda i: (0, 0)),
        ],
        out_specs=pl.BlockSpec((_RPT, _C), lambda i: (i, 0)),
        out_shape=jax.ShapeDtypeStruct((_NP, _C), jnp.float32),
    )(g, dis, W1, b1)


def kernel(x, edge_index, W0, b0, Ws, W1, b1):
    rows = edge_index[1]
    cols = edge_index[0]

    pad = _EPAD - _E
    rows_p = jnp.concatenate([rows, jnp.full((pad,), _N, jnp.int32)])
    cols_p = jnp.concatenate([cols, jnp.zeros((pad,), jnp.int32)])
    rows2 = rows_p.reshape(_NW * _WCH, _CH)
    cols2 = cols_p.reshape(_NW * _WCH, _CH)

    onesg = jnp.ones((_NP, _HP), jnp.float32)
    pdeg0, pdeg1 = _spmm_call(onesg, cols2, rows2)
    d0 = pdeg0[:, :1]
    d1 = pdeg1[:, :1]

    xp = jnp.pad(x, ((0, _NP - _N), (0, 0)))
    b0r = b0.reshape(1, _H)
    b1r = b1.reshape(1, _C)
    x0, g, dis = _pre_call(xp, W0, b0r, d0, d1)

    for l in range(_L):
        beta = jnp.full((1, 1), float(np.log(_THETA / (l + 1) + 1.0)), jnp.float32)
        p0, p1 = _spmm_call(g, cols2, rows2)
        g = _mid_call(p0, p1, g, x0, dis, Ws[l], beta)

    return _head_call(g, dis, W1, b1r)[:_N]


# speedup vs baseline: 5.6394x; 1.0990x over previous
import functools

import numpy as np
import jax
import jax.numpy as jnp
from jax import lax
from jax.experimental import pallas as pl
from jax.experimental.pallas import tpu as pltpu
from jax.experimental.pallas import tpu_sc as plsc

_N = 10000
_E = 320000
_DIN = 128
_H = 64
_HP = 128
_C = 40
_L = 8
_ALPHA = 0.1
_THETA = 0.5

_NC = 2
_NS = 16
_NW = _NC * _NS
_CH = 128
_WCH = 80
_HSL = _WCH // 2
_EPAD = _NW * _WCH * _CH
_RPT = 632
_NP = _RPT * _NS
_GRID = _NS

_RCHUNKS = (0, 128, 256, 384, 504)


def _sc_mesh():
    return plsc.VectorSubcoreMesh(core_axis_name="c", subcore_axis_name="s")


def _fill_iota(buf, base):
    for i in range(_CH // 16):
        buf[pl.ds(16 * i, 16)] = base + 16 * i + lax.iota(jnp.int32, 16)


def _spmm_call(g, cols2, rows2):
    @functools.partial(
        pl.kernel,
        out_type=[
            jax.ShapeDtypeStruct((_NP, _HP), jnp.float32),
            jax.ShapeDtypeStruct((_NP, _HP), jnp.float32),
        ],
        mesh=_sc_mesh(),
        scratch_types=[
            pltpu.VMEM((_HSL, _CH), jnp.int32),
            pltpu.VMEM((_HSL, _CH), jnp.int32),
            pltpu.VMEM((_CH, _HP), jnp.float32),
            pltpu.VMEM((_CH, _HP), jnp.float32),
            pltpu.VMEM((_CH,), jnp.int32),
            pltpu.VMEM_SHARED((_NP, _HP), jnp.float32),
            pltpu.SemaphoreType.DMA,
            pltpu.SemaphoreType.DMA,
            pltpu.SemaphoreType.DMA,
            pltpu.SemaphoreType.DMA,
        ],
    )
    def run(g_hbm, cols_hbm, rows_hbm, outA_hbm, outB_hbm,
            cidx, ridx, gbuf, gbuf1, ibuf, acc, sg0, sg1, ss0, ss1):
        c = lax.axis_index("c")
        s = lax.axis_index("s")
        w = c * _NS + s
        rb = pl.multiple_of(s * _RPT, 8)

        for start in _RCHUNKS:
            _fill_iota(ibuf, rb + start)
            pltpu.sync_copy(g_hbm.at[ibuf], gbuf)
            pltpu.sync_copy(gbuf, acc.at[ibuf])
        plsc.subcore_barrier()

        for half in range(2):
            _fill_iota(ibuf, w * _WCH + half * _HSL)
            pltpu.sync_copy(cols_hbm.at[ibuf.at[pl.ds(0, _HSL)]], cidx)
            pltpu.sync_copy(rows_hbm.at[ibuf.at[pl.ds(0, _HSL)]], ridx)

            pltpu.async_copy(g_hbm.at[cidx.at[0]], gbuf, sg0)
            pltpu.async_copy(g_hbm.at[cidx.at[1]], gbuf1, sg1)

            def step(jj, carry):
                a = jj * 2
                pltpu.make_async_copy(g_hbm.at[cidx.at[a]], gbuf, sg0).wait()
                pltpu.async_copy(gbuf, acc.at[ridx.at[a]], ss0, add=True)
                pltpu.make_async_copy(g_hbm.at[cidx.at[a + 1]], gbuf1, sg1).wait()
                pltpu.async_copy(gbuf1, acc.at[ridx.at[a + 1]], ss1, add=True)
                pltpu.make_async_copy(gbuf, acc.at[ridx.at[a]], ss0).wait()
                pltpu.async_copy(g_hbm.at[cidx.at[a + 2]], gbuf, sg0)
                pltpu.make_async_copy(gbuf1, acc.at[ridx.at[a + 1]], ss1).wait()
                pltpu.async_copy(g_hbm.at[cidx.at[a + 3]], gbuf1, sg1)
                return carry

            lax.fori_loop(0, _HSL // 2 - 1, step, 0)
            a = _HSL - 2
            pltpu.make_async_copy(g_hbm.at[cidx.at[a]], gbuf, sg0).wait()
            pltpu.async_copy(gbuf, acc.at[ridx.at[a]], ss0, add=True)
            pltpu.make_async_copy(g_hbm.at[cidx.at[a + 1]], gbuf1, sg1).wait()
            pltpu.async_copy(gbuf1, acc.at[ridx.at[a + 1]], ss1, add=True)
            pltpu.make_async_copy(gbuf, acc.at[ridx.at[a]], ss0).wait()
            pltpu.make_async_copy(gbuf1, acc.at[ridx.at[a + 1]], ss1).wait()
        plsc.subcore_barrier()

        for start in _RCHUNKS:
            b = pl.multiple_of(rb + start, 8)
            pltpu.sync_copy(acc.at[pl.ds(b, _CH)], gbuf)

            @pl.when(c == 0)
            def _():
                pltpu.sync_copy(gbuf, outA_hbm.at[pl.ds(b, _CH)])

            @pl.when(c == 1)
            def _():
                pltpu.sync_copy(gbuf, outB_hbm.at[pl.ds(b, _CH)])

    return run(g, cols2, rows2)


def _pre_call(x, W0, b0, d0, d1):
    def body(x_ref, w_ref, b_ref, d0_ref, d1_ref, x0_ref, g_ref, dis_ref):
        deg = d0_ref[...] + d1_ref[...] - 1.0
        dis = lax.rsqrt(deg)
        h = jnp.dot(x_ref[...], w_ref[...], preferred_element_type=jnp.float32)
        h = jnp.maximum(h + b_ref[...], 0.0)
        g = dis * h
        x0_ref[...] = h
        g_ref[...] = jnp.concatenate(
            [g, jnp.zeros((_RPT, _HP - _H), jnp.float32)], axis=1)
        dis_ref[...] = dis

    return pl.pallas_call(
        body,
        grid=(_GRID,),
        in_specs=[
            pl.BlockSpec((_RPT, _DIN), lambda i: (i, 0)),
            pl.BlockSpec((_DIN, _H), lambda i: (0, 0)),
            pl.BlockSpec((1, _H), lambda i: (0, 0)),
            pl.BlockSpec((_RPT, 1), lambda i: (i, 0)),
            pl.BlockSpec((_RPT, 1), lambda i: (i, 0)),
        ],
        out_specs=[
            pl.BlockSpec((_RPT, _H), lambda i: (i, 0)),
            pl.BlockSpec((_RPT, _HP), lambda i: (i, 0)),
            pl.BlockSpec((_RPT, 1), lambda i: (i, 0)),
        ],
        out_shape=[
            jax.ShapeDtypeStruct((_NP, _H), jnp.float32),
            jax.ShapeDtypeStruct((_NP, _HP), jnp.float32),
            jax.ShapeDtypeStruct((_NP, 1), jnp.float32),
        ],
    )(x, W0, b0, d0, d1)


def _mid_call(p0, p1, g, x0, dis, Wl, beta):
    def body(p0_ref, p1_ref, g_ref, x0_ref, dis_ref, w_ref, beta_ref, go_ref):
        beta_v = beta_ref[...]
        hp0 = (p0_ref[...] + p1_ref[...] - g_ref[...])[:, :_H]
        hp = dis_ref[...] * hp0
        hpb = (1.0 - _ALPHA) * hp + _ALPHA * x0_ref[...]
        m = jnp.dot(hpb, w_ref[...], preferred_element_type=jnp.float32)
        h = jnp.maximum((1.0 - beta_v) * hpb + beta_v * m, 0.0)
        g = dis_ref[...] * h
        go_ref[...] = jnp.concatenate(
            [g, jnp.zeros((_RPT, _HP - _H), jnp.float32)], axis=1)

    return pl.pallas_call(
        body,
        grid=(_GRID,),
        in_specs=[
            pl.BlockSpec((_RPT, _HP), lambda i: (i, 0)),
            pl.BlockSpec((_RPT, _HP), lambda i: (i, 0)),
            pl.BlockSpec((_RPT, _HP), lambda i: (i, 0)),
            pl.BlockSpec((_RPT, _H), lambda i: (i, 0)),
            pl.BlockSpec((_RPT, 1), lambda i: (i, 0)),
            pl.BlockSpec((_H, _H), lambda i: (0, 0)),
            pl.BlockSpec((1, 1), lambda i: (0, 0)),
        ],
        out_specs=pl.BlockSpec((_RPT, _HP), lambda i: (i, 0)),
        out_shape=jax.ShapeDtypeStruct((_NP, _HP), jnp.float32),
    )(p0, p1, g, x0, dis, Wl, beta)


def _head_call(g, dis, W1, b1):
    def body(g_ref, dis_ref, w_ref, b_ref, o_ref):
        h = g_ref[...][:, :_H] / dis_ref[...]
        z = jnp.dot(h, w_ref[...], preferred_element_type=jnp.float32)
        z = z + b_ref[...]
        zm = z - jnp.max(z, axis=1, keepdims=True)
        o_ref[...] = zm - jnp.log(jnp.sum(jnp.exp(zm), axis=1, keepdims=True))

    return pl.pallas_call(
        body,
        grid=(_GRID,),
        in_specs=[
            pl.BlockSpec((_RPT, _HP), lambda i: (i, 0)),
            pl.BlockSpec((_RPT, 1), lambda i: (i, 0)),
            pl.BlockSpec((_H, _C), lambda i: (0, 0)),
            pl.BlockSpec((1, _C), lambda i: (0, 0)),
        ],
        out_specs=pl.BlockSpec((_RPT, _C), lambda i: (i, 0)),
        out_shape=jax.ShapeDtypeStruct((_NP, _C), jnp.float32),
    )(g, dis, W1, b1)


def kernel(x, edge_index, W0, b0, Ws, W1, b1):
    rows = edge_index[1]
    cols = edge_index[0]

    pad = _EPAD - _E
    rows_p = jnp.concatenate([rows, jnp.full((pad,), _N, jnp.int32)])
    cols_p = jnp.concatenate([cols, jnp.zeros((pad,), jnp.int32)])
    rows2 = rows_p.reshape(_NW * _WCH, _CH)
    cols2 = cols_p.reshape(_NW * _WCH, _CH)

    onesg = jnp.ones((_NP, _HP), jnp.float32)
    pdeg0, pdeg1 = _spmm_call(onesg, cols2, rows2)
    d0 = pdeg0[:, :1]
    d1 = pdeg1[:, :1]

    xp = jnp.pad(x, ((0, _NP - _N), (0, 0)))
    b0r = b0.reshape(1, _H)
    b1r = b1.reshape(1, _C)
    x0, g, dis = _pre_call(xp, W0, b0r, d0, d1)

    for l in range(_L):
        beta = jnp.full((1, 1), float(np.log(_THETA / (l + 1) + 1.0)), jnp.float32)
        p0, p1 = _spmm_call(g, cols2, rows2)
        g = _mid_call(p0, p1, g, x0, dis, Ws[l], beta)

    return _head_call(g, dis, W1, b1r)[:_N]
